# baseline (speedup 1.0000x reference)
import functools

import jax
import jax.numpy as jnp
from jax import lax
from jax.experimental import pallas as pl
from jax.experimental.pallas import tpu as pltpu
from jax.experimental.pallas import tpu_sc as plsc

_B = 16
_N = 16384
_K = 2048
_L = 16
_STEPS = _N // _L


def _fps_sc(pts_hbm, out_hbm, px_v, py_v, pz_v, dist_v, stage_v, sem):
    wid = lax.axis_index("s") * 2 + lax.axis_index("c")

    @pl.when(wid < _B)
    def _run():
        pltpu.sync_copy(pts_hbm.at[0, wid], px_v.at[pl.ds(0, _N)])
        pltpu.sync_copy(pts_hbm.at[1, wid], py_v.at[pl.ds(0, _N)])
        pltpu.sync_copy(pts_hbm.at[2, wid], pz_v.at[pl.ds(0, _N)])

        inf16 = jnp.full((_L,), jnp.inf, jnp.float32)

        def init_body(s, _):
            dist_v[pl.ds(s * _L, _L)] = inf16
            return 0

        lax.fori_loop(0, _STEPS, init_body, 0)

        iota16 = lax.iota(jnp.int32, _L)
        cx0 = px_v[pl.ds(0, _L)][0]
        cy0 = py_v[pl.ds(0, _L)][0]
        cz0 = pz_v[pl.ds(0, _L)][0]

        def body(i, carry):
            cx, cy, cz, stage = carry

            def step(s, sc):
                m16, idx16 = sc
                sl = pl.ds(s * _L, _L)
                dx = px_v[sl] - cx
                dy = py_v[sl] - cy
                dz = pz_v[sl] - cz
                d = dx * dx + dy * dy + dz * dz
                dist = jnp.minimum(dist_v[sl], d)
                dist_v[sl] = dist
                upd = dist > m16
                m16 = jnp.where(upd, dist, m16)
                idx16 = jnp.where(upd, iota16 + s * _L, idx16)
                return (m16, idx16)

            neg16 = jnp.full((_L,), -jnp.inf, jnp.float32)
            m16, idx16 = lax.fori_loop(
                0, _STEPS, step, (neg16, jnp.zeros((_L,), jnp.int32)))
            m = jnp.max(m16)
            cand = jnp.where(m16 == m, idx16, jnp.int32(_N))
            bidx = jnp.min(cand)
            bidx16 = jnp.full((_L,), bidx, jnp.int32)
            base = pl.multiple_of((bidx // _L) * _L, _L)
            sel = iota16 == bidx - base
            ncx = jnp.min(jnp.where(sel, px_v[pl.ds(base, _L)], inf16))
            ncy = jnp.min(jnp.where(sel, py_v[pl.ds(base, _L)], inf16))
            ncz = jnp.min(jnp.where(sel, pz_v[pl.ds(base, _L)], inf16))
            c = i + 1
            slot = lax.rem(c, _L)
            stage = jnp.where(iota16 == slot, bidx16, stage)

            @pl.when(slot == _L - 1)
            def _flush():
                stage_v[...] = stage
                obase = pl.multiple_of((c // _L) * _L, _L)
                pltpu.async_copy(
                    stage_v, out_hbm.at[wid, pl.ds(obase, _L)],
                    sem).wait()

            return (ncx, ncy, ncz, stage)

        lax.fori_loop(0, _K - 1, body,
                      (cx0, cy0, cz0, jnp.zeros((_L,), jnp.int32)))


def kernel(points):
    b, _, n = points.shape
    pts = jnp.transpose(points, (1, 0, 2))
    mesh = plsc.VectorSubcoreMesh(core_axis_name="c", subcore_axis_name="s")
    run = functools.partial(
        pl.kernel,
        mesh=mesh,
        compiler_params=pltpu.CompilerParams(needs_layout_passes=False),
        out_type=jax.ShapeDtypeStruct((b, _K), jnp.int32),
        scratch_types=[
            pltpu.VMEM((n + _L,), jnp.float32),
            pltpu.VMEM((n + _L,), jnp.float32),
            pltpu.VMEM((n + _L,), jnp.float32),
            pltpu.VMEM((n,), jnp.float32),
            pltpu.VMEM((_L,), jnp.int32),
            pltpu.SemaphoreType.DMA,
        ],
    )(_fps_sc)
    return run(pts)


# speedup vs baseline: 10.5114x; 10.5114x over previous
import jax
import jax.numpy as jnp
from jax.experimental import pallas as pl
from jax.experimental.pallas import tpu as pltpu

_K = 2048
_W = 2048


def _fps_kernel(pts_ref, out_ref, dist_ref):
    b, n = dist_ref.shape
    nchunks = n // _W
    dist_ref[...] = jnp.full((b, n), jnp.inf, dtype=jnp.float32)
    cx0 = pts_ref[0, :, 0:1]
    cy0 = pts_ref[1, :, 0:1]
    cz0 = pts_ref[2, :, 0:1]
    stage0 = jnp.zeros((b, 128), jnp.int32)
    lane = jax.lax.broadcasted_iota(jnp.int32, (b, 128), 1)
    iota_f = jax.lax.broadcasted_iota(
        jnp.int32, (b, _W), 1).astype(jnp.float32)

    def body(i, carry):
        cx, cy, cz, stage = carry
        m = jnp.full((b, 1), -jnp.inf, jnp.float32)
        idx = jnp.zeros((b, 1), jnp.int32)
        wx = jnp.zeros((b, 1), jnp.float32)
        wy = jnp.zeros((b, 1), jnp.float32)
        wz = jnp.zeros((b, 1), jnp.float32)
        for j in range(nchunks):
            sl = slice(j * _W, (j + 1) * _W)
            pxc = pts_ref[0, :, sl]
            pyc = pts_ref[1, :, sl]
            pzc = pts_ref[2, :, sl]
            dx = pxc - cx
            dy = pyc - cy
            dz = pzc - cz
            d = dx * dx + dy * dy + dz * dz
            distc = jnp.minimum(dist_ref[:, sl], d)
            dist_ref[:, sl] = distc
            cm = jnp.max(distc, axis=1, keepdims=True)
            lidx = jnp.min(
                jnp.where(distc == cm, iota_f, jnp.float32(_W)),
                axis=1, keepdims=True)
            oh = iota_f == lidx
            ccx = jnp.sum(jnp.where(oh, pxc, jnp.float32(0)), axis=1, keepdims=True)
            ccy = jnp.sum(jnp.where(oh, pyc, jnp.float32(0)), axis=1, keepdims=True)
            ccz = jnp.sum(jnp.where(oh, pzc, jnp.float32(0)), axis=1, keepdims=True)
            better = cm > m
            m = jnp.where(better, cm, m)
            idx = jnp.where(better,
                            lidx.astype(jnp.int32) + jnp.int32(j * _W), idx)
            wx = jnp.where(better, ccx, wx)
            wy = jnp.where(better, ccy, wy)
            wz = jnp.where(better, ccz, wz)
        c = i + 1
        slot = jax.lax.rem(c, 128)
        stage = jnp.where(lane == slot, idx, stage)

        @pl.when(slot == 127)
        def _flush():
            out_ref[jax.lax.div(c, 128)] = stage

        return (wx, wy, wz, stage)

    jax.lax.fori_loop(0, _K - 1, body, (cx0, cy0, cz0, stage0))


def kernel(points):
    b, _, n = points.shape
    pts = jnp.transpose(points, (1, 0, 2))
    out3 = pl.pallas_call(
        _fps_kernel,
        out_shape=jax.ShapeDtypeStruct((_K // 128, b, 128), jnp.int32),
        scratch_shapes=[pltpu.VMEM((b, n), jnp.float32)],
    )(pts)
    return jnp.transpose(out3, (1, 0, 2)).reshape(b, _K)
